# baseline (speedup 1.0000x reference)
import functools

import jax
import jax.numpy as jnp
import numpy as np
from jax.experimental import pallas as pl
from jax.experimental.pallas import tpu as pltpu

B = 4096
C = 10000

MARGIN2 = 0.5
SCALE = 64.0
COS_M2 = float(np.cos(MARGIN2))
SIN_M2 = float(np.sin(MARGIN2))
THRESHOLD = float(np.cos(np.pi - MARGIN2))
THETA_MIN = -2.0

ROWS_PER_BLOCK = 64
NUM_BLOCKS = B // ROWS_PER_BLOCK


def _arcface_block_kernel(y_ref, x_ref, out_ref):
    i = pl.program_id(0)

    x = x_ref[...]
    y = y_ref[...]

    m = jnp.max(x, axis=1)
    v = jnp.sum(y * x, axis=1)
    e = jnp.exp((x - m[:, None]) * SCALE)
    s = jnp.sum(e, axis=1)

    theta = v * COS_M2 - jnp.sqrt(jnp.maximum(1.0 - v * v, 0.0)) * SIN_M2
    w = jnp.where(v > THRESHOLD, theta, THETA_MIN - theta)
    zv = jnp.exp((v - m) * SCALE)
    zw = jnp.exp((w - m) * SCALE)
    s1 = s - zv + zw
    loss = -((w - m) * SCALE - jnp.log(s1))

    part = (jnp.sum(loss) * (1.0 / B)).reshape(1, 1)

    @pl.when(i == 0)
    def _():
        out_ref[...] = part

    @pl.when(i != 0)
    def _():
        out_ref[...] += part


@jax.jit
def kernel(y_true, norm_logits):
    out = pl.pallas_call(
        _arcface_block_kernel,
        grid=(NUM_BLOCKS,),
        in_specs=[
            pl.BlockSpec((ROWS_PER_BLOCK, C), lambda i: (i, 0)),
            pl.BlockSpec((ROWS_PER_BLOCK, C), lambda i: (i, 0)),
        ],
        out_specs=pl.BlockSpec((1, 1), lambda i: (0, 0)),
        out_shape=jax.ShapeDtypeStruct((1, 1), jnp.float32),
    )(y_true, norm_logits)
    return out[0, 0]


# speedup vs baseline: 4.0615x; 4.0615x over previous
import jax
import jax.numpy as jnp
import numpy as np
from jax.experimental import pallas as pl

B = 4096
C = 10000

MARGIN2 = 0.5
SCALE = 64.0
COS_M2 = float(np.cos(MARGIN2))
SIN_M2 = float(np.sin(MARGIN2))
THRESHOLD = float(np.cos(np.pi - MARGIN2))
THETA_MIN = -2.0

COLS_PER_BLOCK = 256
NUM_BLOCKS = B // COLS_PER_BLOCK


def _arcface_block_kernel(y_ref, x_ref, out_ref):
    i = pl.program_id(0)

    x = x_ref[...]
    y = y_ref[...]

    m = jnp.max(x, axis=0)
    v = jnp.sum(y * x, axis=0)
    s = jnp.sum(jnp.exp((x - m[None, :]) * SCALE), axis=0)

    theta = v * COS_M2 - jnp.sqrt(jnp.maximum(1.0 - v * v, 0.0)) * SIN_M2
    w = jnp.where(v > THRESHOLD, theta, THETA_MIN - theta)
    zv = jnp.exp((v - m) * SCALE)
    zw = jnp.exp((w - m) * SCALE)
    s1 = s - zv + zw
    loss = -((w - m) * SCALE - jnp.log(s1))

    part = (jnp.sum(loss) * (1.0 / B)).reshape(1, 1)

    @pl.when(i == 0)
    def _():
        out_ref[...] = part

    @pl.when(i != 0)
    def _():
        out_ref[...] += part


@jax.jit
def kernel(y_true, norm_logits):
    yt = y_true.T
    xt = norm_logits.T
    out = pl.pallas_call(
        _arcface_block_kernel,
        grid=(NUM_BLOCKS,),
        in_specs=[
            pl.BlockSpec((C, COLS_PER_BLOCK), lambda i: (0, i)),
            pl.BlockSpec((C, COLS_PER_BLOCK), lambda i: (0, i)),
        ],
        out_specs=pl.BlockSpec((1, 1), lambda i: (0, 0)),
        out_shape=jax.ShapeDtypeStruct((1, 1), jnp.float32),
    )(yt, xt)
    return out[0, 0]
